# baseline (speedup 1.0000x reference)
import functools

import jax
import jax.numpy as jnp
from jax import lax
from jax.experimental import pallas as pl
from jax.experimental.pallas import tpu as pltpu
from jax.experimental.pallas import tpu_sc as plsc

CHUNK = 128
K = 8


def _build_gather(n_chunks: int, d: int, nw: int):
    per_w = n_chunks // nw
    groups = per_w // K
    mesh = plsc.VectorSubcoreMesh(core_axis_name="c", subcore_axis_name="s")

    @functools.partial(
        pl.kernel,
        mesh=mesh,
        out_type=jax.ShapeDtypeStruct((n_chunks, CHUNK, d), jnp.float32),
        scratch_types=[
            pltpu.VMEM((K, CHUNK), jnp.int32),
            pltpu.VMEM((K, CHUNK, d), jnp.float32),
            pltpu.SemaphoreType.DMA,
        ],
        compiler_params=pltpu.CompilerParams(use_tc_tiling_on_sc=False),
    )
    def gather(idx_hbm, table_hbm, out_hbm, idx_v, rows_v, sem):
        nc = 2
        wid = lax.axis_index("s") * nc + lax.axis_index("c")
        base = wid * per_w

        def body(g, carry):
            row0 = base + g * K
            pltpu.sync_copy(idx_hbm.at[pl.ds(row0, K)], idx_v)
            copies = [
                pltpu.async_copy(table_hbm.at[idx_v.at[j]], rows_v.at[j], sem)
                for j in range(K)
            ]
            for c in copies:
                c.wait()
            pltpu.sync_copy(rows_v, out_hbm.at[pl.ds(row0, K)])
            return carry

        lax.fori_loop(0, groups, body, 0)

    return gather


def kernel(input_ids, wte):
    b0, s = input_ids.shape
    total = b0 * s
    d = wte.shape[1]
    n_chunks = total // CHUNK
    idx = input_ids.reshape(n_chunks, CHUNK).astype(jnp.int32)
    out = _build_gather(n_chunks, d, 32)(idx, wte)
    return out.reshape(b0, s, d)


# speedup vs baseline: 1.0177x; 1.0177x over previous
import functools

import jax
import jax.numpy as jnp
from jax import lax
from jax.experimental import pallas as pl
from jax.experimental.pallas import tpu as pltpu
from jax.experimental.pallas import tpu_sc as plsc

CHUNK = 128
K = 5
NBUF = 2


def _build_gather(n_chunks: int, d: int, nc: int, ns: int):
    nw = nc * ns
    per_w = n_chunks // nw
    v_total = per_w // K
    t_total = v_total // NBUF
    mesh = plsc.VectorSubcoreMesh(core_axis_name="c", subcore_axis_name="s")

    @functools.partial(
        pl.kernel,
        mesh=mesh,
        out_type=jax.ShapeDtypeStruct((n_chunks, CHUNK, d), jnp.float32),
        scratch_types=[
            pltpu.VMEM((NBUF, K, CHUNK), jnp.int32),
            pltpu.VMEM((NBUF, K, CHUNK, d), jnp.float32),
            pltpu.SemaphoreType.DMA,
            pltpu.SemaphoreType.DMA,
            pltpu.SemaphoreType.DMA,
            pltpu.SemaphoreType.DMA,
        ],
        compiler_params=pltpu.CompilerParams(use_tc_tiling_on_sc=False),
    )
    def gather(idx_hbm, table_hbm, out_hbm, idx_v, rows_v, gs0, gs1, ss0, ss1):
        wid = lax.axis_index("s") * nc + lax.axis_index("c")
        base = wid * per_w
        g_sems = (gs0, gs1)
        s_sems = (ss0, ss1)

        def fire(b):
            for j in range(K):
                pltpu.async_copy(
                    table_hbm.at[idx_v.at[b, j]], rows_v.at[b, j], g_sems[b]
                )

        def drain(b):
            for j in range(K):
                pltpu.make_async_copy(
                    table_hbm.at[idx_v.at[b, j]], rows_v.at[b, j], g_sems[b]
                ).wait()

        pltpu.sync_copy(idx_hbm.at[pl.ds(base, K)], idx_v.at[0])

        def outer(t, carry):
            for b in range(NBUF):
                bp = 1 - b
                row0 = base + (t * NBUF + b) * K

                @pl.when(t >= 1)
                def _wait_store():
                    pltpu.make_async_copy(
                        rows_v.at[b],
                        out_hbm.at[pl.ds(row0 - NBUF * K, K)],
                        s_sems[b],
                    ).wait()

                fire(b)

                def _drain_store():
                    drain(bp)
                    pltpu.async_copy(
                        rows_v.at[bp], out_hbm.at[pl.ds(row0 - K, K)], s_sems[bp]
                    )

                if b == 0:
                    pl.when(t >= 1)(_drain_store)
                else:
                    _drain_store()

                def _load_idx():
                    pltpu.sync_copy(
                        idx_hbm.at[pl.ds(row0 + K, K)], idx_v.at[bp]
                    )

                if b == NBUF - 1:
                    pl.when(t < t_total - 1)(_load_idx)
                else:
                    _load_idx()
            return carry

        lax.fori_loop(0, t_total, outer, 0)

        last_row0 = base + (v_total - 1) * K
        drain(1)
        pltpu.async_copy(rows_v.at[1], out_hbm.at[pl.ds(last_row0, K)], ss1)
        pltpu.make_async_copy(
            rows_v.at[0], out_hbm.at[pl.ds(last_row0 - K, K)], ss0
        ).wait()
        pltpu.make_async_copy(
            rows_v.at[1], out_hbm.at[pl.ds(last_row0, K)], ss1
        ).wait()

    return gather


def kernel(input_ids, wte):
    b0, s = input_ids.shape
    total = b0 * s
    d = wte.shape[1]
    n_chunks = total // CHUNK
    idx = input_ids.reshape(n_chunks, CHUNK).astype(jnp.int32)
    out = _build_gather(n_chunks, d, 2, 16)(idx, wte)
    return out.reshape(b0, s, d)


# speedup vs baseline: 1.0183x; 1.0005x over previous
import functools

import jax
import jax.numpy as jnp
from jax import lax
from jax.experimental import pallas as pl
from jax.experimental.pallas import tpu as pltpu
from jax.experimental.pallas import tpu_sc as plsc

G = 4
NBUF = 2


def _build_gather(n_rows: int, seq: int, d: int, nc: int, ns: int):
    nw = nc * ns
    per_w = n_rows // nw
    v_total = per_w // G
    t_total = v_total // NBUF
    splits = [(0, 128), (128, seq - 128)] if seq > 128 else [(0, seq)]
    mesh = plsc.VectorSubcoreMesh(core_axis_name="c", subcore_axis_name="s")

    @functools.partial(
        pl.kernel,
        mesh=mesh,
        out_type=jax.ShapeDtypeStruct((n_rows, seq, d), jnp.float32),
        scratch_types=[
            pltpu.VMEM((NBUF, G, seq), jnp.int32),
            pltpu.VMEM((NBUF, G, seq, d), jnp.float32),
            pltpu.SemaphoreType.DMA,
            pltpu.SemaphoreType.DMA,
            pltpu.SemaphoreType.DMA,
            pltpu.SemaphoreType.DMA,
        ],
        compiler_params=pltpu.CompilerParams(use_tc_tiling_on_sc=False),
    )
    def gather(idx_hbm, table_hbm, out_hbm, idx_v, rows_v, gs0, gs1, ss0, ss1):
        wid = lax.axis_index("s") * nc + lax.axis_index("c")
        base = wid * per_w
        g_sems = (gs0, gs1)
        s_sems = (ss0, ss1)

        def fire(b):
            for i in range(G):
                for off, ln in splits:
                    pltpu.async_copy(
                        table_hbm.at[idx_v.at[b, i, pl.ds(off, ln)]],
                        rows_v.at[b, i, pl.ds(off, ln)],
                        g_sems[b],
                    )

        def drain(b):
            for i in range(G):
                for off, ln in splits:
                    pltpu.make_async_copy(
                        table_hbm.at[idx_v.at[b, i, pl.ds(off, ln)]],
                        rows_v.at[b, i, pl.ds(off, ln)],
                        g_sems[b],
                    ).wait()

        pltpu.sync_copy(idx_hbm.at[pl.ds(base, G)], idx_v.at[0])

        def outer(t, carry):
            for b in range(NBUF):
                bp = 1 - b
                row0 = base + (t * NBUF + b) * G

                @pl.when(t >= 1)
                def _wait_store():
                    pltpu.make_async_copy(
                        rows_v.at[b],
                        out_hbm.at[pl.ds(row0 - NBUF * G, G)],
                        s_sems[b],
                    ).wait()

                fire(b)

                def _drain_store():
                    drain(bp)
                    pltpu.async_copy(
                        rows_v.at[bp], out_hbm.at[pl.ds(row0 - G, G)], s_sems[bp]
                    )

                if b == 0:
                    pl.when(t >= 1)(_drain_store)
                else:
                    _drain_store()

                def _load_idx():
                    pltpu.sync_copy(
                        idx_hbm.at[pl.ds(row0 + G, G)], idx_v.at[bp]
                    )

                if b == NBUF - 1:
                    pl.when(t < t_total - 1)(_load_idx)
                else:
                    _load_idx()
            return carry

        lax.fori_loop(0, t_total, outer, 0)

        last_row0 = base + (v_total - 1) * G
        drain(1)
        pltpu.async_copy(rows_v.at[1], out_hbm.at[pl.ds(last_row0, G)], ss1)
        pltpu.make_async_copy(
            rows_v.at[0], out_hbm.at[pl.ds(last_row0 - G, G)], ss0
        ).wait()
        pltpu.make_async_copy(
            rows_v.at[1], out_hbm.at[pl.ds(last_row0, G)], ss1
        ).wait()

    return gather


def kernel(input_ids, wte):
    n_rows, seq = input_ids.shape
    d = wte.shape[1]
    out = _build_gather(n_rows, seq, d, 2, 16)(input_ids, wte)
    return out


# speedup vs baseline: 1.5574x; 1.5295x over previous
import functools

import jax
import jax.numpy as jnp
from jax import lax
from jax.experimental import pallas as pl
from jax.experimental.pallas import tpu as pltpu
from jax.experimental.pallas import tpu_sc as plsc

G = 4
NBUF = 2


def _build_gather(n_rows: int, seq: int, d: int, nc: int, ns: int):
    nw = nc * ns
    per_w = n_rows // nw
    v_total = per_w // G
    t_total = v_total // NBUF
    splits = [(0, 128), (128, seq - 128)] if seq > 128 else [(0, seq)]
    mesh = plsc.VectorSubcoreMesh(core_axis_name="c", subcore_axis_name="s")

    @functools.partial(
        pl.kernel,
        mesh=mesh,
        out_type=jax.ShapeDtypeStruct((n_rows, seq, d), jnp.float32),
        scratch_types=[
            pltpu.VMEM((NBUF, G, seq), jnp.int32),
            pltpu.VMEM((NBUF, G, seq, d), jnp.float32),
            pltpu.SemaphoreType.DMA,
            pltpu.SemaphoreType.DMA,
            pltpu.SemaphoreType.DMA,
            pltpu.SemaphoreType.DMA,
        ],
        compiler_params=pltpu.CompilerParams(use_tc_tiling_on_sc=False),
    )
    def gather(idx_hbm, table_hbm, out_hbm, idx_v, rows_v, gs0, gs1, ss0, ss1):
        wid = lax.axis_index("s") * nc + lax.axis_index("c")
        base = wid * per_w
        g_sems = (gs0, gs1)
        s_sems = (ss0, ss1)

        def fire(b):
            for i in range(G):
                for off, ln in splits:
                    pltpu.async_copy(
                        table_hbm.at[idx_v.at[b, i, pl.ds(off, ln)]],
                        rows_v.at[b, i, pl.ds(off, ln)],
                        g_sems[b],
                    )

        def drain(b):
            for i in range(G):
                for off, ln in splits:
                    pltpu.make_async_copy(
                        table_hbm.at[idx_v.at[b, i, pl.ds(off, ln)]],
                        rows_v.at[b, i, pl.ds(off, ln)],
                        g_sems[b],
                    ).wait()

        pltpu.sync_copy(idx_hbm.at[pl.ds(base, G)], idx_v.at[0])

        def outer(t, carry):
            for b in range(NBUF):
                bp = 1 - b
                row0 = base + (t * NBUF + b) * G

                @pl.when(t >= 1)
                def _wait_store():
                    pltpu.make_async_copy(
                        rows_v.at[b],
                        out_hbm.at[pl.ds(row0 - NBUF * G, G)],
                        s_sems[b],
                    ).wait()

                fire(b)

                def _drain_store():
                    drain(bp)
                    pltpu.async_copy(
                        rows_v.at[bp], out_hbm.at[pl.ds(row0 - G, G)], s_sems[bp]
                    )

                if b == 0:
                    pl.when(t >= 1)(_drain_store)
                else:
                    _drain_store()

                def _load_idx():
                    pltpu.sync_copy(
                        idx_hbm.at[pl.ds(row0 + G, G)], idx_v.at[bp]
                    )

                if b == NBUF - 1:
                    pl.when(t < t_total - 1)(_load_idx)
                else:
                    _load_idx()
            return carry

        lax.fori_loop(0, t_total, outer, 0)

        last_row0 = base + (v_total - 1) * G
        drain(1)
        pltpu.async_copy(rows_v.at[1], out_hbm.at[pl.ds(last_row0, G)], ss1)
        pltpu.make_async_copy(
            rows_v.at[0], out_hbm.at[pl.ds(last_row0 - G, G)], ss0
        ).wait()
        pltpu.make_async_copy(
            rows_v.at[1], out_hbm.at[pl.ds(last_row0, G)], ss1
        ).wait()

    return gather


def kernel(input_ids, wte):
    from jax.experimental.layout import Layout, with_layout_constraint

    n_rows, seq = input_ids.shape
    d = wte.shape[1]
    ids_c = with_layout_constraint(input_ids, Layout((0, 1), ((8,),)))
    wte_c = with_layout_constraint(wte, Layout((0, 1), ((8,),)))
    out = _build_gather(n_rows, seq, d, 2, 16)(ids_c, wte_c)
    return with_layout_constraint(out, Layout((0, 1, 2), ((8,),)))
